# baseline (speedup 1.0000x reference)
import functools
from math import pi as _PI

import jax
import jax.numpy as jnp
from jax import lax
from jax.experimental import pallas as pl
from jax.experimental.pallas import tpu as pltpu
from jax.experimental.pallas import tpu_sc as plsc

_N = 10000
_E = 320000
_D = 128
_H = 128
_NB = 64
_CUT = 10.0

_NC = 2
_NS = 16
_NW = _NC * _NS
_EPT = _E // _NW
_GC = 80
_NGC = _EPT // _GC
_DC = 2000
_NP = 10240
_NPT = _NP // _NS

_BE = 512
_GE = _E // _BE
_BM = 1280
_GM = _E // _BM
_BN = 1000
_GN = _N // _BN


def _mesh():
    return plsc.VectorSubcoreMesh(core_axis_name="c", subcore_axis_name="s")


def _edge_dots(px, py, pz, ii, jj, kk, pp):
    @functools.partial(
        pl.kernel,
        out_type=jax.ShapeDtypeStruct((_E * 16,), jnp.float32),
        mesh=_mesh(),
        compiler_params=pltpu.CompilerParams(needs_layout_passes=False),
        scratch_types=[
            pltpu.VMEM((_N,), jnp.float32),
            pltpu.VMEM((_N,), jnp.float32),
            pltpu.VMEM((_N,), jnp.float32),
            pltpu.VMEM((_DC,), jnp.int32),
            pltpu.VMEM((_DC,), jnp.int32),
            pltpu.VMEM((_DC,), jnp.int32),
            pltpu.VMEM((_DC,), jnp.int32),
            pltpu.VMEM((_DC * 16,), jnp.float32),
        ],
    )
    def k(px_h, py_h, pz_h, i_h, j_h, k_h, p_h, out_h, pxv, pyv, pzv,
          ivv, jvv, kvv, pvv, outv):
        wid = lax.axis_index("c") * _NS + lax.axis_index("s")
        base = wid * _EPT
        pltpu.sync_copy(px_h, pxv)
        pltpu.sync_copy(py_h, pyv)
        pltpu.sync_copy(pz_h, pzv)
        lane = lax.iota(jnp.int32, 16)

        def chunk_body(cc, carry):
            ebase = base + cc * _DC
            for src, dst in ((i_h, ivv), (j_h, jvv), (k_h, kvv), (p_h, pvv)):
                pltpu.sync_copy(src.at[pl.ds(ebase, _DC)], dst)

            def iter_body(t, c2):
                l = t * 16
                iv = ivv[pl.ds(l, 16)]
                jv = jvv[pl.ds(l, 16)]
                kv = kvv[pl.ds(l, 16)]
                pv = pvv[pl.ds(l, 16)]
                xi = plsc.load_gather(pxv, [iv]); yi = plsc.load_gather(pyv, [iv]); zi = plsc.load_gather(pzv, [iv])
                xj = plsc.load_gather(pxv, [jv]); yj = plsc.load_gather(pyv, [jv]); zj = plsc.load_gather(pzv, [jv])
                xk = plsc.load_gather(pxv, [kv]); yk = plsc.load_gather(pyv, [kv]); zk = plsc.load_gather(pzv, [kv])
                xp = plsc.load_gather(pxv, [pv]); yp = plsc.load_gather(pyv, [pv]); zp = plsc.load_gather(pzv, [pv])
                ax, ay, az = xj - xi, yj - yi, zj - zi
                bx, by, bz = xk - xj, yk - yj, zk - zj
                cx, cy, cz = xp - xj, yp - yj, zp - zj
                da = ax * ax + ay * ay + az * az
                db = bx * bx + by * by + bz * bz
                dc = cx * cx + cy * cy + cz * cz
                de = ax * bx + ay * by + az * bz
                df = ax * cx + ay * cy + az * cz
                u1x = ay * bz - az * by
                u1y = az * bx - ax * bz
                u1z = ax * by - ay * bx
                u2x = ay * cz - az * cy
                u2y = az * cx - ax * cz
                u2z = ax * cy - ay * cx
                n1 = u1x * u1x + u1y * u1y + u1z * u1z
                n2 = u2x * u2x + u2y * u2y + u2z * u2z
                d12 = u1x * u2x + u1y * u2y + u1z * u2z
                wx = u1y * u2z - u1z * u2y
                wy = u1z * u2x - u1x * u2z
                wz = u1x * u2y - u1y * u2x
                nv = wx * wx + wy * wy + wz * wz
                fl = wx * ax + wy * ay + wz * az
                row16 = (l + lane) * 16
                for col, val in enumerate((da, db, dc, de, df, n1, n2, nv, d12, fl)):
                    plsc.store_scatter(outv, [row16 + col], val)
                return c2

            lax.fori_loop(0, _DC // 16, iter_body, 0)
            pltpu.sync_copy(outv, out_h.at[pl.ds(ebase * 16, _DC * 16)])
            return carry

        lax.fori_loop(0, _EPT // _DC, chunk_body, 0)

    return k(px, py, pz, ii, jj, kk, pp)


def _gather4(h, ii, jj, kk, pp):
    dt = h.dtype
    out_t = tuple(jax.ShapeDtypeStruct((_E, _H), dt) for _ in range(4))
    gc = 128
    nfull = _EPT // gc
    tail = _EPT - nfull * gc

    @functools.partial(
        pl.kernel,
        out_type=out_t,
        mesh=_mesh(),
        scratch_types=(
            [pltpu.VMEM((_EPT,), jnp.int32)] * 4
            + [pltpu.VMEM((gc, _H), dt)] * 4
            + [pltpu.SemaphoreType.DMA] * 8
        ),
    )
    def k(h_hbm, i_h, j_h, k_h, p_h, oi, oj, ok, op,
          iv, jv, kv, pv, r0, r1, r2, r3,
          g0, g1, g2, g3, w0, w1, w2, w3):
        wid = lax.axis_index("c") * _NS + lax.axis_index("s")
        base = wid * _EPT
        idxs = (iv, jv, kv, pv)
        rows = (r0, r1, r2, r3)
        gsems = (g0, g1, g2, g3)
        wsems = (w0, w1, w2, w3)
        outs = (oi, oj, ok, op)
        for src, dst in zip((i_h, j_h, k_h, p_h), idxs):
            pltpu.sync_copy(src.at[pl.ds(base, _EPT)], dst)

        for s in range(4):
            pltpu.async_copy(h_hbm.at[idxs[s].at[pl.ds(0, gc)]], rows[s], gsems[s])

        def body(c, carry):
            lo = pl.multiple_of(c * gc, 8)
            go = pl.multiple_of(base + c * gc, 8)
            lo2 = pl.multiple_of((c + 1) * gc, 8)
            for s in range(4):
                pltpu.make_async_copy(h_hbm.at[idxs[s].at[pl.ds(lo, gc)]],
                                      rows[s], gsems[s]).wait()
                pltpu.async_copy(rows[s], outs[s].at[pl.ds(go, gc)], wsems[s])
            for s in range(4):
                pltpu.make_async_copy(rows[s], outs[s].at[pl.ds(go, gc)],
                                      wsems[s]).wait()

                @pl.when(c + 1 < nfull)
                def _(s=s, lo2=lo2):
                    pltpu.async_copy(h_hbm.at[idxs[s].at[pl.ds(lo2, gc)]],
                                     rows[s], gsems[s])
            return carry

        lax.fori_loop(0, nfull, body, 0)
        lo = nfull * gc
        go = base + lo
        gd = [pltpu.async_copy(h_hbm.at[idxs[s].at[pl.ds(lo, tail)]],
                               rows[s].at[pl.ds(0, tail)], gsems[s]) for s in range(4)]
        for d in gd:
            d.wait()
        wd = [pltpu.async_copy(rows[s].at[pl.ds(0, tail)],
                               outs[s].at[pl.ds(go, tail)], wsems[s]) for s in range(4)]
        for d in wd:
            d.wait()

    return k(h, ii, jj, kk, pp)


def _scatter_add(m, ii, zeros):
    gc = 128
    nfull = _EPT // gc
    tail = _EPT - nfull * gc
    npair = nfull // 2

    @functools.partial(
        pl.kernel,
        out_type=jax.ShapeDtypeStruct((_NC, _NP, _H), jnp.float32),
        mesh=_mesh(),
        scratch_types=[
            pltpu.VMEM((gc,), jnp.int32),
            pltpu.VMEM((gc,), jnp.int32),
            pltpu.VMEM((tail,), jnp.int32),
            pltpu.VMEM((gc, _H), jnp.float32),
            pltpu.VMEM((gc, _H), jnp.float32),
            pltpu.VMEM_SHARED((_NP, _H), jnp.float32),
            pltpu.SemaphoreType.DMA,
            pltpu.SemaphoreType.DMA,
        ],
    )
    def k(m_hbm, i_h, z_hbm, out_hbm, idxa, idxb, idxt, ma, mb, acc_sh, sa, sb):
        cid = lax.axis_index("c")
        sid = lax.axis_index("s")
        nb = sid * _NPT
        base = (cid * _NS + sid) * _EPT
        pltpu.sync_copy(z_hbm.at[pl.ds(nb, _NPT)], acc_sh.at[pl.ds(nb, _NPT)])
        plsc.subcore_barrier()

        def _mrows(c):
            return m_hbm.at[pl.ds(pl.multiple_of(base + c * gc, 8), gc)]

        def _irows(c):
            return i_h.at[pl.ds(pl.multiple_of(base + c * gc, 8), gc)]

        pltpu.async_copy(_mrows(0), ma, sa)
        pltpu.async_copy(_irows(0), idxa, sa)

        def body(i, carry):
            c0 = i * 2
            pltpu.make_async_copy(_mrows(c0), ma, sa).wait()
            pltpu.make_async_copy(_irows(c0), idxa, sa).wait()
            pltpu.async_copy(_mrows(c0 + 1), mb, sb)
            pltpu.async_copy(_irows(c0 + 1), idxb, sb)
            pltpu.sync_copy(ma, acc_sh.at[idxa], add=True)
            pltpu.make_async_copy(_mrows(c0 + 1), mb, sb).wait()
            pltpu.make_async_copy(_irows(c0 + 1), idxb, sb).wait()

            @pl.when(c0 + 2 < nfull)
            def _():
                pltpu.async_copy(_mrows(c0 + 2), ma, sa)
                pltpu.async_copy(_irows(c0 + 2), idxa, sa)

            pltpu.sync_copy(mb, acc_sh.at[idxb], add=True)
            return carry

        lax.fori_loop(0, npair, body, 0)
        to = base + nfull * gc
        pltpu.sync_copy(m_hbm.at[pl.ds(to, tail)], ma.at[pl.ds(0, tail)])
        pltpu.sync_copy(i_h.at[pl.ds(to, tail)], idxt)
        pltpu.sync_copy(ma.at[pl.ds(0, tail)], acc_sh.at[idxt], add=True)
        plsc.subcore_barrier()
        pltpu.sync_copy(acc_sh.at[pl.ds(nb, _NPT)], out_hbm.at[cid, pl.ds(nb, _NPT)])

    return k(m, ii, zeros)


def _embed(x, w1, b1, w2, b2):
    def body(x_ref, w1_ref, b1_ref, w2_ref, b2_ref, o_ref):
        t = jnp.dot(x_ref[...], w1_ref[...], preferred_element_type=jnp.float32) + b1_ref[...]
        t = jnp.maximum(t, 0.0)
        o_ref[...] = jnp.dot(t, w2_ref[...], preferred_element_type=jnp.float32) + b2_ref[...]

    return pl.pallas_call(
        body,
        grid=(_GN,),
        in_specs=[
            pl.BlockSpec((_BN, _D), lambda g: (g, 0)),
            pl.BlockSpec((_D, _H), lambda g: (0, 0)),
            pl.BlockSpec((1, _H), lambda g: (0, 0)),
            pl.BlockSpec((_H, _H), lambda g: (0, 0)),
            pl.BlockSpec((1, _H), lambda g: (0, 0)),
        ],
        out_specs=pl.BlockSpec((_BN, _H), lambda g: (g, 0)),
        out_shape=jax.ShapeDtypeStruct((_N, _H), jnp.float32),
    )(x, w1, b1, w2, b2)


def _smear(v, num, stop):
    step = stop / (num - 1)
    coeff = -0.5 / (step * step)
    offs = lax.broadcasted_iota(jnp.int32, (_BE, num), 1).astype(jnp.float32) * step
    d = v - offs
    return jnp.exp(coeff * d * d)


def _features(dots):
    def body(dots_ref, o_ref):
        dt = dots_ref[...]
        s = jnp.sqrt(dt[:, 0:8])
        d1 = s[:, 0:1]; d2 = s[:, 1:2]; d3 = s[:, 2:3]
        y = s[:, 5:8]
        xq = jnp.concatenate([dt[:, 3:5], dt[:, 8:9]], axis=1)
        ang = jnp.arctan2(y, xq)
        t1 = ang[:, 0:1]
        t2 = ang[:, 1:2]
        phi1 = ang[:, 2:3] * jnp.sign(dt[:, 9:10])
        de1 = _smear(d1, 50, _CUT)
        de2 = _smear(d2, 50, _CUT)
        de3 = _smear(d3, 50, _CUT)
        te1 = _smear(t1, 6, _PI)
        te2 = _smear(t2, 6, _PI)
        pe1 = _smear(phi1, 12, 2 * _PI)
        pad = jnp.zeros((_BE, 2), jnp.float32)
        o_ref[...] = jnp.concatenate(
            [de1, de2, te1, de3, te2, pe1, pad], axis=1).astype(jnp.bfloat16)

    return pl.pallas_call(
        body,
        grid=(_GE,),
        in_specs=[pl.BlockSpec((_BE, 16), lambda g: (g, 0))],
        out_specs=pl.BlockSpec((_BE, 176), lambda g: (g, 0)),
        out_shape=jax.ShapeDtypeStruct((_E, 176), jnp.bfloat16),
    )(dots)


def _blockdiag_g(blk):
    wd1a, bd1a = blk["d1a"]
    wd2a, bd2a = blk["d2a"]
    wd3a, bd3a = blk["d3a"]
    z = jnp.zeros
    r0 = jnp.concatenate([wd1a, z((50, 256), jnp.float32)], 1)
    r1 = jnp.concatenate([z((56, 128), jnp.float32), wd2a, z((56, 128), jnp.float32)], 1)
    r2 = jnp.concatenate([z((68, 256), jnp.float32), wd3a], 1)
    wg = jnp.concatenate([r0, r1, r2, z((2, 384), jnp.float32)], 0)
    bg = jnp.concatenate([bd1a, bd2a, bd3a], 0)
    return wg.astype(jnp.bfloat16), jnp.reshape(bg, (1, 384))


def _edge_mlp(feat, hi, hj, hk, hp, blk):
    wg, bg = blk["g"]
    wd1b, bd1b = blk["d1b"]
    wd2b, bd2b = blk["d2b"]
    wd3b, bd3b = blk["d3b"]
    wc1, bc1 = blk["c1"]; wc2, bc2 = blk["c2"]; wc3, bc3 = blk["c3"]

    def body(feat_ref, hi_ref, hj_ref, hk_ref, hp_ref,
             wg_r, bg_r,
             wd1b_r, bd1b_r,
             wd2b_r, bd2b_r,
             wd3b_r, bd3b_r,
             wc1_r, bc1_r, wc2_r, bc2_r, wc3_r, bc3_r, o_ref):
        bf = jnp.bfloat16
        f32 = jnp.float32
        f = feat_ref[...]
        gh = jnp.maximum(jnp.dot(f, wg_r[...], preferred_element_type=f32) + bg_r[...], 0.0)
        gh = gh.astype(bf)
        g1 = jnp.dot(gh[:, 0:128], wd1b_r[...], preferred_element_type=f32) + bd1b_r[...]
        g2 = jnp.dot(gh[:, 128:256], wd2b_r[...], preferred_element_type=f32) + bd2b_r[...]
        g3 = jnp.dot(gh[:, 256:384], wd3b_r[...], preferred_element_type=f32) + bd3b_r[...]

        m1 = jnp.dot(hi_ref[...].astype(bf), wc1_r[0:128, :], preferred_element_type=f32)
        m1 += jnp.dot(hj_ref[...].astype(bf), wc1_r[128:256, :], preferred_element_type=f32)
        m1 += jnp.dot(hk_ref[...].astype(bf), wc1_r[256:384, :], preferred_element_type=f32)
        m1 += jnp.dot(hp_ref[...].astype(bf), wc1_r[384:512, :], preferred_element_type=f32)
        m1 += jnp.dot(g1.astype(bf), wc1_r[512:640, :], preferred_element_type=f32)
        m1 += jnp.dot(g2.astype(bf), wc1_r[640:768, :], preferred_element_type=f32)
        m1 += jnp.dot(g3.astype(bf), wc1_r[768:896, :], preferred_element_type=f32)
        m1 += bc1_r[...]
        m2 = jnp.dot(jnp.maximum(m1, 0.0).astype(bf), wc2_r[...], preferred_element_type=f32) + bc2_r[...]
        o_ref[...] = jnp.dot(jnp.maximum(m2, 0.0).astype(bf), wc3_r[...], preferred_element_type=f32) + bc3_r[...]

    cfix = lambda g: (0, 0)
    return pl.pallas_call(
        body,
        grid=(_GM,),
        in_specs=[
            pl.BlockSpec((_BM, 176), lambda g: (g, 0)),
            pl.BlockSpec((_BM, _H), lambda g: (g, 0)),
            pl.BlockSpec((_BM, _H), lambda g: (g, 0)),
            pl.BlockSpec((_BM, _H), lambda g: (g, 0)),
            pl.BlockSpec((_BM, _H), lambda g: (g, 0)),
            pl.BlockSpec((176, 384), cfix), pl.BlockSpec((1, 384), cfix),
            pl.BlockSpec((_H, _H), cfix), pl.BlockSpec((1, _H), cfix),
            pl.BlockSpec((_H, _H), cfix), pl.BlockSpec((1, _H), cfix),
            pl.BlockSpec((_H, _H), cfix), pl.BlockSpec((1, _H), cfix),
            pl.BlockSpec((896, 512), cfix), pl.BlockSpec((1, 512), cfix),
            pl.BlockSpec((512, 256), cfix), pl.BlockSpec((1, 256), cfix),
            pl.BlockSpec((256, _H), cfix), pl.BlockSpec((1, _H), cfix),
        ],
        out_specs=pl.BlockSpec((_BM, _H), lambda g: (g, 0)),
        out_shape=jax.ShapeDtypeStruct((_E, _H), jnp.float32),
    )(feat, hi, hj, hk, hp,
      wg, bg, wd1b, bd1b, wd2b, bd2b, wd3b, bd3b,
      wc1, bc1, wc2, bc2, wc3, bc3)


def _add3(h, p0, p1):
    def body(a_ref, b_ref, c_ref, o_ref):
        o_ref[...] = a_ref[...] + b_ref[...] + c_ref[...]

    return pl.pallas_call(
        body,
        grid=(_GN,),
        in_specs=[pl.BlockSpec((_BN, _H), lambda g: (g, 0))] * 3,
        out_specs=pl.BlockSpec((_BN, _H), lambda g: (g, 0)),
        out_shape=jax.ShapeDtypeStruct((_N, _H), jnp.float32),
    )(h, p0, p1)


def _segsum(h, batch3):
    def body(h_ref, b_ref, o_ref):
        g = pl.program_id(0)

        @pl.when(g == 0)
        def _():
            o_ref[...] = jnp.zeros_like(o_ref)

        brow = b_ref[0]
        rows = lax.broadcasted_iota(jnp.int32, (_NB, _BN), 0)
        oh = (rows == brow).astype(jnp.float32)
        o_ref[...] += jnp.dot(oh, h_ref[...], preferred_element_type=jnp.float32)

    return pl.pallas_call(
        body,
        grid=(_GN,),
        in_specs=[
            pl.BlockSpec((_BN, _H), lambda g: (g, 0)),
            pl.BlockSpec((1, 1, _BN), lambda g: (g, 0, 0)),
        ],
        out_specs=pl.BlockSpec((_NB, _H), lambda g: (0, 0)),
        out_shape=jax.ShapeDtypeStruct((_NB, _H), jnp.float32),
    )(h, batch3)


def _head(seg, wl1, bl1, wl2, bl2):
    def body(s_ref, w1_ref, b1_ref, w2_ref, b2_ref, o_ref):
        t = jnp.dot(s_ref[...], w1_ref[...], preferred_element_type=jnp.float32) + b1_ref[...]
        t = jnp.maximum(t, 0.0)
        o_ref[...] = jnp.dot(t, w2_ref[...], preferred_element_type=jnp.float32) + b2_ref[...]

    return pl.pallas_call(
        body,
        grid=(1,),
        in_specs=[
            pl.BlockSpec((_NB, _H), lambda g: (0, 0)),
            pl.BlockSpec((_H, _NB), lambda g: (0, 0)),
            pl.BlockSpec((1, _NB), lambda g: (0, 0)),
            pl.BlockSpec((_NB, 1), lambda g: (0, 0)),
            pl.BlockSpec((1, 1), lambda g: (0, 0)),
        ],
        out_specs=pl.BlockSpec((_NB, 1), lambda g: (0, 0)),
        out_shape=jax.ShapeDtypeStruct((_NB, 1), jnp.float32),
    )(seg, wl1, bl1, wl2, bl2)


def _row(b):
    return jnp.reshape(b, (1, -1))


def kernel(x, pos, batch, edge_index_3rd, params):
    ii = edge_index_3rd[0]
    jj = edge_index_3rd[1]
    kk = edge_index_3rd[2]
    pp = edge_index_3rd[3]
    px = jnp.asarray(pos[:, 0])
    py = jnp.asarray(pos[:, 1])
    pz = jnp.asarray(pos[:, 2])

    h = _embed(x, params["emb1"][0], _row(params["emb1"][1]),
               params["emb2"][0], _row(params["emb2"][1]))
    dots = jnp.reshape(_edge_dots(px, py, pz, ii, jj, kk, pp), (_E, 16))
    feat = _features(dots)
    zeros = jnp.zeros((_NP, _H), jnp.float32)

    for blk in params["inter"]:
        hi, hj, hk, hp = _gather4(h, ii, jj, kk, pp)
        bw = {
            "d1b": blk["d1b"], "d2b": blk["d2b"], "d3b": blk["d3b"],
            "c1": blk["c1"], "c2": blk["c2"], "c3": blk["c3"],
        }
        bw = {k_: (w.astype(jnp.bfloat16), _row(b)) for k_, (w, b) in bw.items()}
        bw["g"] = _blockdiag_g(blk)
        m = _edge_mlp(feat, hi, hj, hk, hp, bw)
        parts = _scatter_add(m, ii, zeros)
        h = _add3(h, parts[0, :_N], parts[1, :_N])

    batch3 = jnp.reshape(batch.astype(jnp.int32), (_GN, 1, _BN))
    seg = _segsum(h, batch3)
    return _head(seg, params["lin1"][0], _row(params["lin1"][1]),
                 params["lin2"][0], _row(params["lin2"][1]))


# speedup vs baseline: 1.0088x; 1.0088x over previous
import functools
from math import pi as _PI

import jax
import jax.numpy as jnp
from jax import lax
from jax.experimental import pallas as pl
from jax.experimental.pallas import tpu as pltpu
from jax.experimental.pallas import tpu_sc as plsc

_N = 10000
_E = 320000
_D = 128
_H = 128
_NB = 64
_CUT = 10.0

_NC = 2
_NS = 16
_NW = _NC * _NS
_EPT = _E // _NW
_GC = 80
_NGC = _EPT // _GC
_DC = 2000
_NP = 10240
_NPT = _NP // _NS

_BE = 512
_GE = _E // _BE
_BM = 1280
_GM = _E // _BM
_BN = 1000
_GN = _N // _BN


def _mesh():
    return plsc.VectorSubcoreMesh(core_axis_name="c", subcore_axis_name="s")


def _edge_dots(px, py, pz, ii, jj, kk, pp):
    @functools.partial(
        pl.kernel,
        out_type=jax.ShapeDtypeStruct((_E * 16,), jnp.float32),
        mesh=_mesh(),
        compiler_params=pltpu.CompilerParams(needs_layout_passes=False),
        scratch_types=[
            pltpu.VMEM((_N,), jnp.float32),
            pltpu.VMEM((_N,), jnp.float32),
            pltpu.VMEM((_N,), jnp.float32),
            pltpu.VMEM((_DC,), jnp.int32),
            pltpu.VMEM((_DC,), jnp.int32),
            pltpu.VMEM((_DC,), jnp.int32),
            pltpu.VMEM((_DC,), jnp.int32),
            pltpu.VMEM((_DC * 16,), jnp.float32),
        ],
    )
    def k(px_h, py_h, pz_h, i_h, j_h, k_h, p_h, out_h, pxv, pyv, pzv,
          ivv, jvv, kvv, pvv, outv):
        wid = lax.axis_index("c") * _NS + lax.axis_index("s")
        base = wid * _EPT
        pltpu.sync_copy(px_h, pxv)
        pltpu.sync_copy(py_h, pyv)
        pltpu.sync_copy(pz_h, pzv)
        lane = lax.iota(jnp.int32, 16)

        def chunk_body(cc, carry):
            ebase = base + cc * _DC
            for src, dst in ((i_h, ivv), (j_h, jvv), (k_h, kvv), (p_h, pvv)):
                pltpu.sync_copy(src.at[pl.ds(ebase, _DC)], dst)

            def iter_body(t, c2):
                l = t * 16
                iv = ivv[pl.ds(l, 16)]
                jv = jvv[pl.ds(l, 16)]
                kv = kvv[pl.ds(l, 16)]
                pv = pvv[pl.ds(l, 16)]
                xi = plsc.load_gather(pxv, [iv]); yi = plsc.load_gather(pyv, [iv]); zi = plsc.load_gather(pzv, [iv])
                xj = plsc.load_gather(pxv, [jv]); yj = plsc.load_gather(pyv, [jv]); zj = plsc.load_gather(pzv, [jv])
                xk = plsc.load_gather(pxv, [kv]); yk = plsc.load_gather(pyv, [kv]); zk = plsc.load_gather(pzv, [kv])
                xp = plsc.load_gather(pxv, [pv]); yp = plsc.load_gather(pyv, [pv]); zp = plsc.load_gather(pzv, [pv])
                ax, ay, az = xj - xi, yj - yi, zj - zi
                bx, by, bz = xk - xj, yk - yj, zk - zj
                cx, cy, cz = xp - xj, yp - yj, zp - zj
                da = ax * ax + ay * ay + az * az
                db = bx * bx + by * by + bz * bz
                dc = cx * cx + cy * cy + cz * cz
                de = ax * bx + ay * by + az * bz
                df = ax * cx + ay * cy + az * cz
                u1x = ay * bz - az * by
                u1y = az * bx - ax * bz
                u1z = ax * by - ay * bx
                u2x = ay * cz - az * cy
                u2y = az * cx - ax * cz
                u2z = ax * cy - ay * cx
                n1 = u1x * u1x + u1y * u1y + u1z * u1z
                n2 = u2x * u2x + u2y * u2y + u2z * u2z
                d12 = u1x * u2x + u1y * u2y + u1z * u2z
                wx = u1y * u2z - u1z * u2y
                wy = u1z * u2x - u1x * u2z
                wz = u1x * u2y - u1y * u2x
                nv = wx * wx + wy * wy + wz * wz
                fl = wx * ax + wy * ay + wz * az
                row16 = (l + lane) * 16
                for col, val in enumerate((da, db, dc, de, df, n1, n2, nv, d12, fl)):
                    plsc.store_scatter(outv, [row16 + col], val)
                return c2

            lax.fori_loop(0, _DC // 16, iter_body, 0)
            pltpu.sync_copy(outv, out_h.at[pl.ds(ebase * 16, _DC * 16)])
            return carry

        lax.fori_loop(0, _EPT // _DC, chunk_body, 0)

    return k(px, py, pz, ii, jj, kk, pp)


def _gather4(h, ii, jj, kk, pp):
    dt = h.dtype
    out_t = tuple(jax.ShapeDtypeStruct((_E, _H), dt) for _ in range(4))
    gc = 128
    nfull = _EPT // gc
    tail = _EPT - nfull * gc

    @functools.partial(
        pl.kernel,
        out_type=out_t,
        mesh=_mesh(),
        scratch_types=(
            [pltpu.VMEM((_EPT,), jnp.int32)] * 4
            + [pltpu.VMEM((gc, _H), dt)] * 4
            + [pltpu.SemaphoreType.DMA] * 8
        ),
    )
    def k(h_hbm, i_h, j_h, k_h, p_h, oi, oj, ok, op,
          iv, jv, kv, pv, r0, r1, r2, r3,
          g0, g1, g2, g3, w0, w1, w2, w3):
        wid = lax.axis_index("c") * _NS + lax.axis_index("s")
        base = wid * _EPT
        idxs = (iv, jv, kv, pv)
        rows = (r0, r1, r2, r3)
        gsems = (g0, g1, g2, g3)
        wsems = (w0, w1, w2, w3)
        outs = (oi, oj, ok, op)
        for src, dst in zip((i_h, j_h, k_h, p_h), idxs):
            pltpu.sync_copy(src.at[pl.ds(base, _EPT)], dst)

        for s in range(4):
            pltpu.async_copy(h_hbm.at[idxs[s].at[pl.ds(0, gc)]], rows[s], gsems[s])

        def body(c, carry):
            lo = pl.multiple_of(c * gc, 8)
            go = pl.multiple_of(base + c * gc, 8)
            lo2 = pl.multiple_of((c + 1) * gc, 8)
            for s in range(4):
                pltpu.make_async_copy(h_hbm.at[idxs[s].at[pl.ds(lo, gc)]],
                                      rows[s], gsems[s]).wait()
                pltpu.async_copy(rows[s], outs[s].at[pl.ds(go, gc)], wsems[s])
            for s in range(4):
                pltpu.make_async_copy(rows[s], outs[s].at[pl.ds(go, gc)],
                                      wsems[s]).wait()

                @pl.when(c + 1 < nfull)
                def _(s=s, lo2=lo2):
                    pltpu.async_copy(h_hbm.at[idxs[s].at[pl.ds(lo2, gc)]],
                                     rows[s], gsems[s])
            return carry

        lax.fori_loop(0, nfull, body, 0)
        lo = nfull * gc
        go = base + lo
        gd = [pltpu.async_copy(h_hbm.at[idxs[s].at[pl.ds(lo, tail)]],
                               rows[s].at[pl.ds(0, tail)], gsems[s]) for s in range(4)]
        for d in gd:
            d.wait()
        wd = [pltpu.async_copy(rows[s].at[pl.ds(0, tail)],
                               outs[s].at[pl.ds(go, tail)], wsems[s]) for s in range(4)]
        for d in wd:
            d.wait()

    return k(h, ii, jj, kk, pp)


def _scatter_add(m, ii, zeros):
    gc = 128
    nfull = _EPT // gc
    tail = _EPT - nfull * gc
    npair = nfull // 2

    @functools.partial(
        pl.kernel,
        out_type=jax.ShapeDtypeStruct((_NC, _NP, _H), jnp.float32),
        mesh=_mesh(),
        scratch_types=[
            pltpu.VMEM((gc,), jnp.int32),
            pltpu.VMEM((gc,), jnp.int32),
            pltpu.VMEM((tail,), jnp.int32),
            pltpu.VMEM((gc, _H), jnp.float32),
            pltpu.VMEM((gc, _H), jnp.float32),
            pltpu.VMEM_SHARED((_NP, _H), jnp.float32),
            pltpu.SemaphoreType.DMA,
            pltpu.SemaphoreType.DMA,
        ],
    )
    def k(m_hbm, i_h, z_hbm, out_hbm, idxa, idxb, idxt, ma, mb, acc_sh, sa, sb):
        cid = lax.axis_index("c")
        sid = lax.axis_index("s")
        nb = sid * _NPT
        base = (cid * _NS + sid) * _EPT
        pltpu.sync_copy(z_hbm.at[pl.ds(nb, _NPT)], acc_sh.at[pl.ds(nb, _NPT)])
        plsc.subcore_barrier()

        def _mrows(c):
            return m_hbm.at[pl.ds(pl.multiple_of(base + c * gc, 8), gc)]

        def _irows(c):
            return i_h.at[pl.ds(pl.multiple_of(base + c * gc, 8), gc)]

        pltpu.async_copy(_mrows(0), ma, sa)
        pltpu.async_copy(_irows(0), idxa, sa)

        def body(i, carry):
            c0 = i * 2
            pltpu.make_async_copy(_mrows(c0), ma, sa).wait()
            pltpu.make_async_copy(_irows(c0), idxa, sa).wait()
            pltpu.async_copy(_mrows(c0 + 1), mb, sb)
            pltpu.async_copy(_irows(c0 + 1), idxb, sb)
            pltpu.sync_copy(ma, acc_sh.at[idxa], add=True)
            pltpu.make_async_copy(_mrows(c0 + 1), mb, sb).wait()
            pltpu.make_async_copy(_irows(c0 + 1), idxb, sb).wait()

            @pl.when(c0 + 2 < nfull)
            def _():
                pltpu.async_copy(_mrows(c0 + 2), ma, sa)
                pltpu.async_copy(_irows(c0 + 2), idxa, sa)

            pltpu.sync_copy(mb, acc_sh.at[idxb], add=True)
            return carry

        lax.fori_loop(0, npair, body, 0)
        to = base + nfull * gc
        pltpu.sync_copy(m_hbm.at[pl.ds(to, tail)], ma.at[pl.ds(0, tail)])
        pltpu.sync_copy(i_h.at[pl.ds(to, tail)], idxt)
        pltpu.sync_copy(ma.at[pl.ds(0, tail)], acc_sh.at[idxt], add=True)
        plsc.subcore_barrier()
        pltpu.sync_copy(acc_sh.at[pl.ds(nb, _NPT)], out_hbm.at[cid, pl.ds(nb, _NPT)])

    return k(m, ii, zeros)


def _embed(x, w1, b1, w2, b2):
    def body(x_ref, w1_ref, b1_ref, w2_ref, b2_ref, o_ref):
        t = jnp.dot(x_ref[...], w1_ref[...], preferred_element_type=jnp.float32) + b1_ref[...]
        t = jnp.maximum(t, 0.0)
        o_ref[...] = jnp.dot(t, w2_ref[...], preferred_element_type=jnp.float32) + b2_ref[...]

    return pl.pallas_call(
        body,
        grid=(_GN,),
        in_specs=[
            pl.BlockSpec((_BN, _D), lambda g: (g, 0)),
            pl.BlockSpec((_D, _H), lambda g: (0, 0)),
            pl.BlockSpec((1, _H), lambda g: (0, 0)),
            pl.BlockSpec((_H, _H), lambda g: (0, 0)),
            pl.BlockSpec((1, _H), lambda g: (0, 0)),
        ],
        out_specs=pl.BlockSpec((_BN, _H), lambda g: (g, 0)),
        out_shape=jax.ShapeDtypeStruct((_N, _H), jnp.float32),
    )(x, w1, b1, w2, b2)


def _smear(v, num, stop):
    step = stop / (num - 1)
    coeff = -0.5 / (step * step)
    offs = lax.broadcasted_iota(jnp.int32, (_BE, num), 1).astype(jnp.float32) * step
    d = v - offs
    return jnp.exp(coeff * d * d)


def _features(dots):
    def body(dots_ref, o_ref):
        dt = dots_ref[...]
        s = jnp.sqrt(dt[:, 0:8])
        d1 = s[:, 0:1]; d2 = s[:, 1:2]; d3 = s[:, 2:3]
        y = s[:, 5:8]
        xq = jnp.concatenate([dt[:, 3:5], dt[:, 8:9]], axis=1)
        ang = jnp.arctan2(y, xq)
        t1 = ang[:, 0:1]
        t2 = ang[:, 1:2]
        phi1 = ang[:, 2:3] * jnp.sign(dt[:, 9:10])
        de1 = _smear(d1, 50, _CUT)
        de2 = _smear(d2, 50, _CUT)
        de3 = _smear(d3, 50, _CUT)
        te1 = _smear(t1, 6, _PI)
        te2 = _smear(t2, 6, _PI)
        pe1 = _smear(phi1, 12, 2 * _PI)
        pad = jnp.zeros((_BE, 2), jnp.float32)
        o_ref[...] = jnp.concatenate(
            [de1, de2, te1, de3, te2, pe1, pad], axis=1).astype(jnp.bfloat16)

    return pl.pallas_call(
        body,
        grid=(_GE,),
        in_specs=[pl.BlockSpec((_BE, 16), lambda g: (g, 0))],
        out_specs=pl.BlockSpec((_BE, 176), lambda g: (g, 0)),
        out_shape=jax.ShapeDtypeStruct((_E, 176), jnp.bfloat16),
    )(dots)


def _edge_mlp(feat, hi, hj, hk, hp, blk):
    wd1a, bd1a = blk["d1a"]; wd1b, bd1b = blk["d1b"]
    wd2a, bd2a = blk["d2a"]; wd2b, bd2b = blk["d2b"]
    wd3a, bd3a = blk["d3a"]; wd3b, bd3b = blk["d3b"]
    wc1, bc1 = blk["c1"]; wc2, bc2 = blk["c2"]; wc3, bc3 = blk["c3"]

    def body(feat_ref, hi_ref, hj_ref, hk_ref, hp_ref,
             wd1a_r, bd1a_r, wd1b_r, bd1b_r,
             wd2a_r, bd2a_r, wd2b_r, bd2b_r,
             wd3a_r, bd3a_r, wd3b_r, bd3b_r,
             wc1_r, bc1_r, wc2_r, bc2_r, wc3_r, bc3_r, o_ref):
        bf = jnp.bfloat16
        f32 = jnp.float32
        f = feat_ref[...]
        x1 = f[:, 0:50]
        x2 = f[:, 50:106]
        x3 = f[:, 106:174]
        g1 = jnp.maximum(jnp.dot(x1, wd1a_r[...], preferred_element_type=f32) + bd1a_r[...], 0.0)
        g1 = jnp.dot(g1.astype(bf), wd1b_r[...], preferred_element_type=f32) + bd1b_r[...]
        g2 = jnp.maximum(jnp.dot(x2, wd2a_r[...], preferred_element_type=f32) + bd2a_r[...], 0.0)
        g2 = jnp.dot(g2.astype(bf), wd2b_r[...], preferred_element_type=f32) + bd2b_r[...]
        g3 = jnp.maximum(jnp.dot(x3, wd3a_r[...], preferred_element_type=f32) + bd3a_r[...], 0.0)
        g3 = jnp.dot(g3.astype(bf), wd3b_r[...], preferred_element_type=f32) + bd3b_r[...]

        m1 = jnp.dot(hi_ref[...].astype(bf), wc1_r[0:128, :], preferred_element_type=f32)
        m1 += jnp.dot(hj_ref[...].astype(bf), wc1_r[128:256, :], preferred_element_type=f32)
        m1 += jnp.dot(hk_ref[...].astype(bf), wc1_r[256:384, :], preferred_element_type=f32)
        m1 += jnp.dot(hp_ref[...].astype(bf), wc1_r[384:512, :], preferred_element_type=f32)
        m1 += jnp.dot(g1.astype(bf), wc1_r[512:640, :], preferred_element_type=f32)
        m1 += jnp.dot(g2.astype(bf), wc1_r[640:768, :], preferred_element_type=f32)
        m1 += jnp.dot(g3.astype(bf), wc1_r[768:896, :], preferred_element_type=f32)
        m1 += bc1_r[...]
        m2 = jnp.dot(jnp.maximum(m1, 0.0).astype(bf), wc2_r[...], preferred_element_type=f32) + bc2_r[...]
        o_ref[...] = jnp.dot(jnp.maximum(m2, 0.0).astype(bf), wc3_r[...], preferred_element_type=f32) + bc3_r[...]

    cfix = lambda g: (0, 0)
    return pl.pallas_call(
        body,
        grid=(_GM,),
        in_specs=[
            pl.BlockSpec((_BM, 176), lambda g: (g, 0)),
            pl.BlockSpec((_BM, _H), lambda g: (g, 0)),
            pl.BlockSpec((_BM, _H), lambda g: (g, 0)),
            pl.BlockSpec((_BM, _H), lambda g: (g, 0)),
            pl.BlockSpec((_BM, _H), lambda g: (g, 0)),
            pl.BlockSpec((50, _H), cfix), pl.BlockSpec((1, _H), cfix),
            pl.BlockSpec((_H, _H), cfix), pl.BlockSpec((1, _H), cfix),
            pl.BlockSpec((56, _H), cfix), pl.BlockSpec((1, _H), cfix),
            pl.BlockSpec((_H, _H), cfix), pl.BlockSpec((1, _H), cfix),
            pl.BlockSpec((68, _H), cfix), pl.BlockSpec((1, _H), cfix),
            pl.BlockSpec((_H, _H), cfix), pl.BlockSpec((1, _H), cfix),
            pl.BlockSpec((896, 512), cfix), pl.BlockSpec((1, 512), cfix),
            pl.BlockSpec((512, 256), cfix), pl.BlockSpec((1, 256), cfix),
            pl.BlockSpec((256, _H), cfix), pl.BlockSpec((1, _H), cfix),
        ],
        out_specs=pl.BlockSpec((_BM, _H), lambda g: (g, 0)),
        out_shape=jax.ShapeDtypeStruct((_E, _H), jnp.float32),
    )(feat, hi, hj, hk, hp,
      wd1a, bd1a, wd1b, bd1b, wd2a, bd2a, wd2b, bd2b, wd3a, bd3a, wd3b, bd3b,
      wc1, bc1, wc2, bc2, wc3, bc3)


def _add3(h, p0, p1):
    def body(a_ref, b_ref, c_ref, o_ref):
        o_ref[...] = a_ref[...] + b_ref[...] + c_ref[...]

    return pl.pallas_call(
        body,
        grid=(_GN,),
        in_specs=[pl.BlockSpec((_BN, _H), lambda g: (g, 0))] * 3,
        out_specs=pl.BlockSpec((_BN, _H), lambda g: (g, 0)),
        out_shape=jax.ShapeDtypeStruct((_N, _H), jnp.float32),
    )(h, p0, p1)


def _segsum(h, batch3):
    def body(h_ref, b_ref, o_ref):
        g = pl.program_id(0)

        @pl.when(g == 0)
        def _():
            o_ref[...] = jnp.zeros_like(o_ref)

        brow = b_ref[0]
        rows = lax.broadcasted_iota(jnp.int32, (_NB, _BN), 0)
        oh = (rows == brow).astype(jnp.float32)
        o_ref[...] += jnp.dot(oh, h_ref[...], preferred_element_type=jnp.float32)

    return pl.pallas_call(
        body,
        grid=(_GN,),
        in_specs=[
            pl.BlockSpec((_BN, _H), lambda g: (g, 0)),
            pl.BlockSpec((1, 1, _BN), lambda g: (g, 0, 0)),
        ],
        out_specs=pl.BlockSpec((_NB, _H), lambda g: (0, 0)),
        out_shape=jax.ShapeDtypeStruct((_NB, _H), jnp.float32),
    )(h, batch3)


def _head(seg, wl1, bl1, wl2, bl2):
    def body(s_ref, w1_ref, b1_ref, w2_ref, b2_ref, o_ref):
        t = jnp.dot(s_ref[...], w1_ref[...], preferred_element_type=jnp.float32) + b1_ref[...]
        t = jnp.maximum(t, 0.0)
        o_ref[...] = jnp.dot(t, w2_ref[...], preferred_element_type=jnp.float32) + b2_ref[...]

    return pl.pallas_call(
        body,
        grid=(1,),
        in_specs=[
            pl.BlockSpec((_NB, _H), lambda g: (0, 0)),
            pl.BlockSpec((_H, _NB), lambda g: (0, 0)),
            pl.BlockSpec((1, _NB), lambda g: (0, 0)),
            pl.BlockSpec((_NB, 1), lambda g: (0, 0)),
            pl.BlockSpec((1, 1), lambda g: (0, 0)),
        ],
        out_specs=pl.BlockSpec((_NB, 1), lambda g: (0, 0)),
        out_shape=jax.ShapeDtypeStruct((_NB, 1), jnp.float32),
    )(seg, wl1, bl1, wl2, bl2)


def _row(b):
    return jnp.reshape(b, (1, -1))


def kernel(x, pos, batch, edge_index_3rd, params):
    ii = edge_index_3rd[0]
    jj = edge_index_3rd[1]
    kk = edge_index_3rd[2]
    pp = edge_index_3rd[3]
    px = jnp.asarray(pos[:, 0])
    py = jnp.asarray(pos[:, 1])
    pz = jnp.asarray(pos[:, 2])

    h = _embed(x, params["emb1"][0], _row(params["emb1"][1]),
               params["emb2"][0], _row(params["emb2"][1]))
    dots = jnp.reshape(_edge_dots(px, py, pz, ii, jj, kk, pp), (_E, 16))
    feat = _features(dots)
    zeros = jnp.zeros((_NP, _H), jnp.float32)

    for blk in params["inter"]:
        hi, hj, hk, hp = _gather4(h, ii, jj, kk, pp)
        bw = {
            "d1a": blk["d1a"], "d1b": blk["d1b"],
            "d2a": blk["d2a"], "d2b": blk["d2b"],
            "d3a": blk["d3a"], "d3b": blk["d3b"],
            "c1": blk["c1"], "c2": blk["c2"], "c3": blk["c3"],
        }
        bw = {k_: (w.astype(jnp.bfloat16), _row(b)) for k_, (w, b) in bw.items()}
        m = _edge_mlp(feat, hi, hj, hk, hp, bw)
        parts = _scatter_add(m, ii, zeros)
        h = _add3(h, parts[0, :_N], parts[1, :_N])

    batch3 = jnp.reshape(batch.astype(jnp.int32), (_GN, 1, _BN))
    seg = _segsum(h, batch3)
    return _head(seg, params["lin1"][0], _row(params["lin1"][1]),
                 params["lin2"][0], _row(params["lin2"][1]))
